# Initial kernel scaffold; baseline (speedup 1.0000x reference)
import jax
import jax.numpy as jnp
from jax.experimental import pallas as pl


def kernel(age, trestbps, chol, thalach, oldpeak, slope, ca, thal, emb_table, W1, b1, W2, b2, W3, b3):
    raise NotImplementedError("write your pallas kernel here")


# speedup vs baseline: 5.3259x; 5.3259x over previous
import functools

import jax
import jax.numpy as jnp
import numpy as np
from jax.experimental import pallas as pl

B = 16384
AGE_BOUNDARIES = (18., 25., 30., 35., 40., 45., 50., 55., 60., 65.)
N_BUCKETS = 11
THAL_VOCAB = 3
HASH_BUCKETS = 1000
N_COMBO = N_BUCKETS * THAL_VOCAB

_OFF_AGE = 0
_OFF_AB = 1
_OFF_CROSS = 12
_OFF_CA = 1012
_OFF_CHOL = 1013
_OFF_OLDPEAK = 1014
_OFF_SLOPE = 1015
_OFF_EMB = 1016
_OFF_THAL_OH = 1024
_OFF_THALACH = 1027
_OFF_TRESTBPS = 1028

_DENSE_ROWS = (_OFF_AGE, _OFF_CA, _OFF_CHOL, _OFF_OLDPEAK, _OFF_SLOPE,
               _OFF_THALACH, _OFF_TRESTBPS)


def _crossed_idx(ab: int, th: int) -> int:
    return (ab * 1000003 + th * 7919) % HASH_BUCKETS


def _build_tables_kernel(w1_ref, emb_ref, b1_ref, t_ref, r_ref):
    e = jax.lax.dot_general(emb_ref[...], w1_ref[_OFF_EMB:_OFF_EMB + 8, :],
                            (((1,), (0,)), ((), ())),
                            preferred_element_type=jnp.float32)
    b1 = b1_ref[...]
    rows = []
    for ab in range(N_BUCKETS):
        for th in range(THAL_VOCAB):
            c = _crossed_idx(ab, th)
            rows.append(w1_ref[_OFF_AB + ab, :] + w1_ref[_OFF_CROSS + c, :]
                        + e[th, :] + w1_ref[_OFF_THAL_OH + th, :] + b1)
    t_ref[...] = jnp.stack(rows, axis=0)
    r_ref[...] = jnp.stack([w1_ref[r, :] for r in _DENSE_ROWS], axis=0)


def _mlp_kernel(age_ref, thal_ref, s7_ref, t_ref, r_ref, w2_ref, b2_ref,
                w3_ref, b3_ref, out_ref):
    age = age_ref[...]
    ab = jnp.zeros_like(age, dtype=jnp.int32)
    for bound in AGE_BOUNDARIES:
        ab = ab + (age >= bound).astype(jnp.int32)
    idx = ab * THAL_VOCAB + thal_ref[...]
    lanes = jax.lax.broadcasted_iota(jnp.int32, (idx.shape[0], N_COMBO), 1)
    onehot = (idx == lanes).astype(jnp.float32)
    cat = jax.lax.dot_general(onehot, t_ref[...],
                              (((1,), (0,)), ((), ())),
                              preferred_element_type=jnp.float32)
    dense = jax.lax.dot_general(s7_ref[...], r_ref[...],
                                (((1,), (0,)), ((), ())),
                                preferred_element_type=jnp.float32)
    h1 = jnp.maximum(cat + dense, 0.0)
    h2 = jax.lax.dot_general(h1, w2_ref[...],
                             (((1,), (0,)), ((), ())),
                             preferred_element_type=jnp.float32)
    h2 = jnp.maximum(h2 + b2_ref[...], 0.0)
    o = jax.lax.dot_general(h2, w3_ref[...],
                            (((1,), (0,)), ((), ())),
                            preferred_element_type=jnp.float32)
    o = o + b3_ref[...]
    out_ref[...] = 1.0 / (1.0 + jnp.exp(-o))


def kernel(age, trestbps, chol, thalach, oldpeak, slope, ca, thal,
           emb_table, W1, b1, W2, b2, W3, b3):
    t33, r7 = pl.pallas_call(
        _build_tables_kernel,
        out_shape=(jax.ShapeDtypeStruct((N_COMBO, 128), jnp.float32),
                   jax.ShapeDtypeStruct((7, 128), jnp.float32)),
    )(W1, emb_table, b1)

    s7 = jnp.stack([age, ca, chol, oldpeak, slope, thalach, trestbps], axis=1)
    age2 = age[:, None]
    thal2 = thal[:, None]

    bb = 2048
    grid = B // bb
    out = pl.pallas_call(
        _mlp_kernel,
        grid=(grid,),
        in_specs=[
            pl.BlockSpec((bb, 1), lambda i: (i, 0)),
            pl.BlockSpec((bb, 1), lambda i: (i, 0)),
            pl.BlockSpec((bb, 7), lambda i: (i, 0)),
            pl.BlockSpec((N_COMBO, 128), lambda i: (0, 0)),
            pl.BlockSpec((7, 128), lambda i: (0, 0)),
            pl.BlockSpec((128, 64), lambda i: (0, 0)),
            pl.BlockSpec((1, 64), lambda i: (0, 0)),
            pl.BlockSpec((64, 1), lambda i: (0, 0)),
            pl.BlockSpec((1, 1), lambda i: (0, 0)),
        ],
        out_specs=pl.BlockSpec((bb, 1), lambda i: (i, 0)),
        out_shape=jax.ShapeDtypeStruct((B, 1), jnp.float32),
    )(age2, thal2, s7, t33, r7, W2, b2[None, :], W3, b3[None, :])
    return out


# speedup vs baseline: 19.5749x; 3.6754x over previous
import jax
import jax.numpy as jnp
from jax.experimental import pallas as pl

B = 16384
AGE_BOUNDARIES = (18., 25., 30., 35., 40., 45., 50., 55., 60., 65.)
N_BUCKETS = 11
THAL_VOCAB = 3
HASH_BUCKETS = 1000
N_COMBO = N_BUCKETS * THAL_VOCAB

_OFF_AGE = 0
_OFF_AB = 1
_OFF_CROSS = 12
_OFF_CA = 1012
_OFF_CHOL = 1013
_OFF_OLDPEAK = 1014
_OFF_SLOPE = 1015
_OFF_EMB = 1016
_OFF_THAL_OH = 1024
_OFF_THALACH = 1027
_OFF_TRESTBPS = 1028

_DENSE_ROWS = (_OFF_AGE, _OFF_CA, _OFF_CHOL, _OFF_OLDPEAK, _OFF_SLOPE,
               _OFF_THALACH, _OFF_TRESTBPS)

_TN = (((0,), (0,)), ((), ()))


def _crossed_idx(ab: int, th: int) -> int:
    return (ab * 1000003 + th * 7919) % HASH_BUCKETS


def _build_tables_kernel(w1_ref, emb_ref, b1_ref, t_ref, r_ref):
    e = jax.lax.dot_general(emb_ref[...], w1_ref[_OFF_EMB:_OFF_EMB + 8, :],
                            (((1,), (0,)), ((), ())),
                            preferred_element_type=jnp.float32)
    b1 = b1_ref[...]
    rows = []
    for ab in range(N_BUCKETS):
        for th in range(THAL_VOCAB):
            c = _crossed_idx(ab, th)
            rows.append(w1_ref[_OFF_AB + ab, :] + w1_ref[_OFF_CROSS + c, :]
                        + e[th, :] + w1_ref[_OFF_THAL_OH + th, :] + b1)
    t_ref[...] = jnp.stack(rows, axis=0)
    r_ref[...] = jnp.stack([w1_ref[r, :] for r in _DENSE_ROWS], axis=0)


def _mlp_kernel(age_ref, thal_ref, s7_ref, t_ref, r_ref, w2_ref, b2_ref,
                w3_ref, b3_ref, out_ref):
    age = age_ref[...]
    ab = jnp.zeros_like(age, dtype=jnp.int32)
    for bound in AGE_BOUNDARIES:
        ab = ab + (age >= bound).astype(jnp.int32)
    idx = ab * THAL_VOCAB + thal_ref[...]
    combos = jax.lax.broadcasted_iota(jnp.int32, (N_COMBO, idx.shape[1]), 0)
    onehot_t = (combos == idx).astype(jnp.float32)
    cat_t = jax.lax.dot_general(t_ref[...], onehot_t, _TN,
                                preferred_element_type=jnp.float32)
    dense_t = jax.lax.dot_general(r_ref[...], s7_ref[...], _TN,
                                  preferred_element_type=jnp.float32)
    h1_t = jnp.maximum(cat_t + dense_t, 0.0)
    h2_t = jax.lax.dot_general(w2_ref[...], h1_t, _TN,
                               preferred_element_type=jnp.float32)
    h2_t = jnp.maximum(h2_t + b2_ref[...], 0.0)
    o_t = jax.lax.dot_general(w3_ref[...], h2_t, _TN,
                              preferred_element_type=jnp.float32)
    o_t = o_t + b3_ref[...]
    out_ref[...] = 1.0 / (1.0 + jnp.exp(-o_t))


def kernel(age, trestbps, chol, thalach, oldpeak, slope, ca, thal,
           emb_table, W1, b1, W2, b2, W3, b3):
    t33, r7 = pl.pallas_call(
        _build_tables_kernel,
        out_shape=(jax.ShapeDtypeStruct((N_COMBO, 128), jnp.float32),
                   jax.ShapeDtypeStruct((7, 128), jnp.float32)),
    )(W1, emb_table, b1)

    s7_t = jnp.stack([age, ca, chol, oldpeak, slope, thalach, trestbps],
                     axis=0)
    age_row = age[None, :]
    thal_row = thal[None, :]

    bb = 4096
    grid = B // bb
    out_t = pl.pallas_call(
        _mlp_kernel,
        grid=(grid,),
        in_specs=[
            pl.BlockSpec((1, bb), lambda i: (0, i)),
            pl.BlockSpec((1, bb), lambda i: (0, i)),
            pl.BlockSpec((7, bb), lambda i: (0, i)),
            pl.BlockSpec((N_COMBO, 128), lambda i: (0, 0)),
            pl.BlockSpec((7, 128), lambda i: (0, 0)),
            pl.BlockSpec((128, 64), lambda i: (0, 0)),
            pl.BlockSpec((64, 1), lambda i: (0, 0)),
            pl.BlockSpec((64, 1), lambda i: (0, 0)),
            pl.BlockSpec((1, 1), lambda i: (0, 0)),
        ],
        out_specs=pl.BlockSpec((1, bb), lambda i: (0, i)),
        out_shape=jax.ShapeDtypeStruct((1, B), jnp.float32),
    )(age_row, thal_row, s7_t, t33, r7, W2, b2[:, None], W3, b3[:, None])
    return out_t.reshape(B, 1)
